# baseline (speedup 1.0000x reference)
import dataclasses

import jax
import jax.numpy as jnp
from jax import lax
from jax.experimental import pallas as pl
from jax.experimental.pallas import tpu as pltpu
from jax.experimental.pallas import tpu_sc as plsc

N = 2048
D = 512
E = 32768
NN = N * N
NTEC = 16
ET = E // NTEC
CH = 512
SPDUMP = CH * N
DUMPW = NTEC * 1024
NWIN = 33
NENT = NWIN * 128
DIAG_PER_TEC = CH // NTEC


def _rsqrt16(x):
    i = plsc.bitcast(x, jnp.int32)
    i = 0x5F3759DF - lax.shift_right_logical(i, 1)
    y = plsc.bitcast(i, jnp.float32)
    for _ in range(3):
        y = y * (1.5 - 0.5 * x * y * y)
    return y


def _p_body(x_ref, w_ref, b_ref, o_ref):
    xb = x_ref[...].astype(jnp.bfloat16).astype(jnp.float32)
    wb = w_ref[...].astype(jnp.bfloat16).astype(jnp.float32)
    o_ref[...] = jnp.sum(xb * wb, axis=1, keepdims=True) + b_ref[...]


def _init_body(o_ref):
    i = pl.program_id(0)
    r = lax.broadcasted_iota(jnp.int32, (CH, N), 0) + i * CH
    c = lax.broadcasted_iota(jnp.int32, (CH, N), 1)
    o_ref[...] = jnp.where(r == c, 1.0, 0.0).astype(jnp.float32)


def _sc_body(edges_ref, p_ref, out_ref,
             u_buf, v_buf, val_buf, p_loc, dinv_loc, dacc, red_loc, dred,
             lidx_b, gidx_b, val_b, acc_b, z2d,
             dstage, dinv_sh, spbuf):
    core = lax.axis_index("c")
    s = lax.axis_index("s")
    iota16 = lax.iota(jnp.int32, 16)
    zeros16 = jnp.zeros((16,), jnp.float32)

    pltpu.sync_copy(edges_ref.at[pl.ds(s * ET, ET)], u_buf)
    pltpu.sync_copy(edges_ref.at[pl.ds(E + s * ET, ET)], v_buf)
    pltpu.sync_copy(p_ref, p_loc)

    @pl.loop(0, N // 16)
    def _(i):
        dacc[pl.ds(i * 16, 16)] = zeros16

    @pl.loop(0, NENT // 16)
    def _(i):
        z2d[pl.ds(i * 16, 16)] = zeros16

    @pl.loop(0, ET // 16)
    def _(i):
        sl = pl.ds(i * 16, 16)
        u16 = u_buf[sl]
        v16 = v_buf[sl]
        pu = plsc.load_gather(p_loc, [u16])
        pv = plsc.load_gather(p_loc, [v16])
        valr = jnp.maximum(pu - pv, 0.0)
        val_buf[sl] = valr
        w = 1.0 + valr
        plsc.addupdate_scatter(dacc, [u16], w)
        plsc.addupdate_scatter(dacc, [v16], w)

    for b in range(NTEC):
        pltpu.sync_copy(dacc.at[pl.ds(b * 128, 128)], dstage.at[b, s])
    plsc.subcore_barrier()
    pltpu.sync_copy(dstage.at[s], red_loc)
    for k in range(8):
        acc = red_loc[0, pl.ds(k * 16, 16)]
        for t in range(1, NTEC):
            acc = acc + red_loc[t, pl.ds(k * 16, 16)]
        dred[pl.ds(k * 16, 16)] = _rsqrt16(acc + 1.0)
    pltpu.sync_copy(dred, dinv_sh.at[pl.ds(s * 128, 128)])
    plsc.subcore_barrier()
    pltpu.sync_copy(dinv_sh, dinv_loc)

    for j in range(2):
        chunk = core * 2 + j
        rowbase = chunk * CH

        for t in range(8):
            k16 = iota16 + (t & 1) * 16
            n16 = rowbase + s * DIAG_PER_TEC + k16
            dn = plsc.load_gather(dinv_loc, [n16])
            vv = -dn * dn if t < 2 else zeros16
            sl = pl.ds(2 * ET + t * 16, 16)
            lidx_b[sl] = (s * DIAG_PER_TEC + k16) * N + n16
            gidx_b[sl] = NN + n16 * (N + 1)
            val_b[sl] = vv

        @pl.loop(0, ET // 16)
        def _(i):
            sl = pl.ds(i * 16, 16)
            u16 = u_buf[sl]
            v16 = v_buf[sl]
            valr = val_buf[sl]
            zero = valr == 0.0
            w = 1.0 + valr
            du = plsc.load_gather(dinv_loc, [u16])
            dv = plsc.load_gather(dinv_loc, [v16])
            duv = du * dv
            for (r16, c16, vv, base) in (
                (u16, jnp.where(zero, u16, v16), -w * jnp.where(zero, du * du, duv), 0),
                (v16, jnp.where(zero, v16, u16), -w * jnp.where(zero, dv * dv, duv), ET),
            ):
                inm = lax.shift_right_logical(r16, 9) == chunk
                e16 = iota16 + i * 16 + base
                lidx = jnp.where(inm, (r16 - rowbase) * N + c16,
                                 SPDUMP + s * 1024 + (e16 & 1023))
                gdump = 2 * NN + ((core * 2 + j) * NTEC + s) * NENT + e16
                gidx = jnp.where(inm, NN + r16 * N + c16, gdump)
                sl2 = pl.ds(base + i * 16, 16)
                lidx_b[sl2] = lidx
                gidx_b[sl2] = gidx
                val_b[sl2] = vv

        pltpu.sync_copy(z2d, spbuf.at[lidx_b])
        plsc.subcore_barrier()
        pltpu.sync_copy(val_b, spbuf.at[lidx_b], add=True)
        plsc.subcore_barrier()
        pltpu.sync_copy(spbuf.at[lidx_b], acc_b)
        plsc.subcore_barrier()

        @pl.loop(0, NENT // 16)
        def _(i):
            sl = pl.ds(i * 16, 16)
            lidx = lidx_b[sl]
            accv = acc_b[sl]
            val_b[sl] = jnp.where(lidx >= SPDUMP, 0.0, accv)

        pltpu.sync_copy(val_b, out_ref.at[gidx_b])


def kernel(featuers, Edges, W_e1, b_e1):
    p2 = pl.pallas_call(
        _p_body,
        out_shape=jax.ShapeDtypeStruct((N, 1), jnp.float32),
    )(featuers, W_e1.reshape(1, D), b_e1.reshape(1, 1))
    p = p2.reshape(N)

    init2d = pl.pallas_call(
        _init_body,
        grid=(4 * N // CH,),
        out_specs=pl.BlockSpec((CH, N), lambda i: (i, 0)),
        out_shape=jax.ShapeDtypeStruct((4 * N, N), jnp.float32),
    )()
    init = init2d.reshape(4 * NN)

    cp = pltpu.CompilerParams()
    if "needs_layout_passes" in pltpu.CompilerParams.__dataclass_fields__:
        cp = dataclasses.replace(cp, needs_layout_passes=False)
    mesh = plsc.VectorSubcoreMesh(core_axis_name="c", subcore_axis_name="s")
    sc_kern = pl.kernel(
        _sc_body,
        out_type=(),
        mesh=mesh,
        compiler_params=cp,
        scratch_types=[
            pltpu.VMEM((ET,), jnp.int32),
            pltpu.VMEM((ET,), jnp.int32),
            pltpu.VMEM((ET,), jnp.float32),
            pltpu.VMEM((N,), jnp.float32),
            pltpu.VMEM((N,), jnp.float32),
            pltpu.VMEM((N,), jnp.float32),
            pltpu.VMEM((NTEC, 128), jnp.float32),
            pltpu.VMEM((128,), jnp.float32),
            pltpu.VMEM((NENT,), jnp.int32),
            pltpu.VMEM((NENT,), jnp.int32),
            pltpu.VMEM((NENT,), jnp.float32),
            pltpu.VMEM((NENT,), jnp.float32),
            pltpu.VMEM((NENT,), jnp.float32),
            pltpu.VMEM_SHARED((NTEC, NTEC, 128), jnp.float32),
            pltpu.VMEM_SHARED((N,), jnp.float32),
            pltpu.VMEM_SHARED((SPDUMP + DUMPW,), jnp.float32),
        ],
    )

    out_ref = jax.new_ref(init)
    sc_kern(Edges.reshape(2 * E), p, out_ref)
    return out_ref[...].reshape(2, 2, N, N)


# speedup vs baseline: 1.0021x; 1.0021x over previous
import dataclasses

import jax
import jax.numpy as jnp
from jax import lax
from jax.experimental import pallas as pl
from jax.experimental.pallas import tpu as pltpu
from jax.experimental.pallas import tpu_sc as plsc

N = 2048
D = 512
E = 32768
NN = N * N
NTEC = 16
ET = E // NTEC
CH = 512
SPDUMP = CH * N
DUMPW = NTEC * 1024
NWIN = 33
NENT = NWIN * 128
DIAG_PER_TEC = CH // NTEC


def _rsqrt16(x):
    i = plsc.bitcast(x, jnp.int32)
    i = 0x5F3759DF - lax.shift_right_logical(i, 1)
    y = plsc.bitcast(i, jnp.float32)
    for _ in range(3):
        y = y * (1.5 - 0.5 * x * y * y)
    return y


def _p_body(x_ref, w_ref, b_ref, o_ref):
    xb = x_ref[...].astype(jnp.bfloat16).astype(jnp.float32)
    wb = w_ref[...].astype(jnp.bfloat16).astype(jnp.float32)
    o_ref[...] = jnp.sum(xb * wb, axis=1, keepdims=True) + b_ref[...]


def _init_body(o_ref):
    i = pl.program_id(0)
    r = lax.broadcasted_iota(jnp.int32, (CH, N), 0) + i * CH
    c = lax.broadcasted_iota(jnp.int32, (CH, N), 1)
    o_ref[...] = jnp.where(r == c, 1.0, 0.0).astype(jnp.float32)


def _sc_body(edges_ref, p_ref, out_ref,
             u_buf, v_buf, val_buf, p_loc, dinv_loc, dacc, red_loc, dred,
             lidx_b, gidx_b, val_b, acc_b, z2d,
             dstage, dinv_sh, spbuf):
    core = lax.axis_index("c")
    s = lax.axis_index("s")
    iota16 = lax.iota(jnp.int32, 16)
    zeros16 = jnp.zeros((16,), jnp.float32)

    with jax.named_scope("ph_stage"):
        pltpu.sync_copy(edges_ref.at[pl.ds(s * ET, ET)], u_buf)
        pltpu.sync_copy(edges_ref.at[pl.ds(E + s * ET, ET)], v_buf)
        pltpu.sync_copy(p_ref, p_loc)

        @pl.loop(0, N // 16)
        def _(i):
            dacc[pl.ds(i * 16, 16)] = zeros16

        @pl.loop(0, NENT // 16)
        def _(i):
            z2d[pl.ds(i * 16, 16)] = zeros16

    with jax.named_scope("ph_degree"):
        @pl.loop(0, ET // 16)
        def _(i):
            sl = pl.ds(i * 16, 16)
            u16 = u_buf[sl]
            v16 = v_buf[sl]
            pu = plsc.load_gather(p_loc, [u16])
            pv = plsc.load_gather(p_loc, [v16])
            valr = jnp.maximum(pu - pv, 0.0)
            val_buf[sl] = valr
            w = 1.0 + valr
            plsc.addupdate_scatter(dacc, [u16], w)
            plsc.addupdate_scatter(dacc, [v16], w)

    with jax.named_scope("ph_dreduce"):
        for b in range(NTEC):
            pltpu.sync_copy(dacc.at[pl.ds(b * 128, 128)], dstage.at[b, s])
        plsc.subcore_barrier()
        pltpu.sync_copy(dstage.at[s], red_loc)
        for k in range(8):
            acc = red_loc[0, pl.ds(k * 16, 16)]
            for t in range(1, NTEC):
                acc = acc + red_loc[t, pl.ds(k * 16, 16)]
            dred[pl.ds(k * 16, 16)] = _rsqrt16(acc + 1.0)
        pltpu.sync_copy(dred, dinv_sh.at[pl.ds(s * 128, 128)])
        plsc.subcore_barrier()
        pltpu.sync_copy(dinv_sh, dinv_loc)

    for j in range(2):
        chunk = core * 2 + j
        rowbase = chunk * CH

        for t in range(8):
            k16 = iota16 + (t & 1) * 16
            n16 = rowbase + s * DIAG_PER_TEC + k16
            dn = plsc.load_gather(dinv_loc, [n16])
            vv = -dn * dn if t < 2 else zeros16
            sl = pl.ds(2 * ET + t * 16, 16)
            lidx_b[sl] = (s * DIAG_PER_TEC + k16) * N + n16
            gidx_b[sl] = NN + n16 * (N + 1)
            val_b[sl] = vv

        gen_scope = jax.named_scope("ph_gen")
        gen_scope.__enter__()

        @pl.loop(0, ET // 16)
        def _(i):
            sl = pl.ds(i * 16, 16)
            u16 = u_buf[sl]
            v16 = v_buf[sl]
            valr = val_buf[sl]
            zero = valr == 0.0
            w = 1.0 + valr
            du = plsc.load_gather(dinv_loc, [u16])
            dv = plsc.load_gather(dinv_loc, [v16])
            duv = du * dv
            for (r16, c16, vv, base) in (
                (u16, jnp.where(zero, u16, v16), -w * jnp.where(zero, du * du, duv), 0),
                (v16, jnp.where(zero, v16, u16), -w * jnp.where(zero, dv * dv, duv), ET),
            ):
                inm = lax.shift_right_logical(r16, 9) == chunk
                e16 = iota16 + i * 16 + base
                lidx = jnp.where(inm, (r16 - rowbase) * N + c16,
                                 SPDUMP + s * 1024 + (e16 & 1023))
                gdump = 2 * NN + ((core * 2 + j) * NTEC + s) * NENT + e16
                gidx = jnp.where(inm, NN + r16 * N + c16, gdump)
                sl2 = pl.ds(base + i * 16, 16)
                lidx_b[sl2] = lidx
                gidx_b[sl2] = gidx
                val_b[sl2] = vv

        gen_scope.__exit__(None, None, None)

        with jax.named_scope("ph_spzero"):
            pltpu.sync_copy(z2d, spbuf.at[lidx_b])
            plsc.subcore_barrier()
        with jax.named_scope("ph_spadd"):
            pltpu.sync_copy(val_b, spbuf.at[lidx_b], add=True)
            plsc.subcore_barrier()
        with jax.named_scope("ph_spgather"):
            pltpu.sync_copy(spbuf.at[lidx_b], acc_b)
            plsc.subcore_barrier()

        with jax.named_scope("ph_select"):
            @pl.loop(0, NENT // 16)
            def _(i):
                sl = pl.ds(i * 16, 16)
                lidx = lidx_b[sl]
                accv = acc_b[sl]
                val_b[sl] = jnp.where(lidx >= SPDUMP, 0.0, accv)

        with jax.named_scope("ph_hbm"):
            pltpu.sync_copy(val_b, out_ref.at[gidx_b])


def kernel(featuers, Edges, W_e1, b_e1):
    p2 = pl.pallas_call(
        _p_body,
        out_shape=jax.ShapeDtypeStruct((N, 1), jnp.float32),
    )(featuers, W_e1.reshape(1, D), b_e1.reshape(1, 1))
    p = p2.reshape(N)

    init2d = pl.pallas_call(
        _init_body,
        grid=(4 * N // CH,),
        out_specs=pl.BlockSpec((CH, N), lambda i: (i, 0)),
        out_shape=jax.ShapeDtypeStruct((4 * N, N), jnp.float32),
    )()
    init = init2d.reshape(4 * NN)

    cp = pltpu.CompilerParams()
    if "needs_layout_passes" in pltpu.CompilerParams.__dataclass_fields__:
        cp = dataclasses.replace(cp, needs_layout_passes=False)
    mesh = plsc.VectorSubcoreMesh(core_axis_name="c", subcore_axis_name="s")
    sc_kern = pl.kernel(
        _sc_body,
        out_type=(),
        mesh=mesh,
        compiler_params=cp,
        scratch_types=[
            pltpu.VMEM((ET,), jnp.int32),
            pltpu.VMEM((ET,), jnp.int32),
            pltpu.VMEM((ET,), jnp.float32),
            pltpu.VMEM((N,), jnp.float32),
            pltpu.VMEM((N,), jnp.float32),
            pltpu.VMEM((N,), jnp.float32),
            pltpu.VMEM((NTEC, 128), jnp.float32),
            pltpu.VMEM((128,), jnp.float32),
            pltpu.VMEM((NENT,), jnp.int32),
            pltpu.VMEM((NENT,), jnp.int32),
            pltpu.VMEM((NENT,), jnp.float32),
            pltpu.VMEM((NENT,), jnp.float32),
            pltpu.VMEM((NENT,), jnp.float32),
            pltpu.VMEM_SHARED((NTEC, NTEC, 128), jnp.float32),
            pltpu.VMEM_SHARED((N,), jnp.float32),
            pltpu.VMEM_SHARED((SPDUMP + DUMPW,), jnp.float32),
        ],
    )

    out_ref = jax.new_ref(init)
    sc_kern(Edges.reshape(2 * E), p, out_ref)
    return out_ref[...].reshape(2, 2, N, N)


# speedup vs baseline: 3.5850x; 3.5775x over previous
import dataclasses

import jax
import jax.numpy as jnp
from jax import lax
from jax.experimental import pallas as pl
from jax.experimental.pallas import tpu as pltpu
from jax.experimental.pallas import tpu_sc as plsc

N = 2048
D = 512
E = 32768
NN = N * N
NTEC = 16
ET = E // NTEC
CH = 512
SPDUMP = CH * N
DUMPW = NTEC * 1024
NWIN = 33
NENT = NWIN * 128
DIAG_PER_TEC = CH // NTEC


def _rsqrt16(x):
    i = plsc.bitcast(x, jnp.int32)
    i = 0x5F3759DF - lax.shift_right_logical(i, 1)
    y = plsc.bitcast(i, jnp.float32)
    for _ in range(3):
        y = y * (1.5 - 0.5 * x * y * y)
    return y


def _p_body(x_ref, w_ref, b_ref, o_ref):
    xb = x_ref[...].astype(jnp.bfloat16).astype(jnp.float32)
    wb = w_ref[...].astype(jnp.bfloat16).astype(jnp.float32)
    o_ref[...] = jnp.sum(xb * wb, axis=1, keepdims=True) + b_ref[...]


def _init_body(o_ref):
    i = pl.program_id(0)
    blk = jnp.where(i < 4, i, i + 4)
    r = lax.broadcasted_iota(jnp.int32, (CH, N), 0) + blk * CH
    c = lax.broadcasted_iota(jnp.int32, (CH, N), 1)
    o_ref[...] = jnp.where(r == c, 1.0, 0.0).astype(jnp.float32)


def _sc_body(edges_ref, p_ref, out_ref,
             u_buf, v_buf, val_buf, p_loc, dinv_loc, dacc, red_loc, dred,
             lidx_b, val_b,
             dstage, dinv_sh, spbuf):
    core = lax.axis_index("c")
    s = lax.axis_index("s")
    iota16 = lax.iota(jnp.int32, 16)
    zeros16 = jnp.zeros((16,), jnp.float32)

    with jax.named_scope("ph_stage"):
        pltpu.sync_copy(edges_ref.at[pl.ds(s * ET, ET)], u_buf)
        pltpu.sync_copy(edges_ref.at[pl.ds(E + s * ET, ET)], v_buf)
        pltpu.sync_copy(p_ref, p_loc)

        @pl.loop(0, N // 16)
        def _(i):
            dacc[pl.ds(i * 16, 16)] = zeros16

    with jax.named_scope("ph_degree"):
        @pl.loop(0, ET // 16)
        def _(i):
            sl = pl.ds(i * 16, 16)
            u16 = u_buf[sl]
            v16 = v_buf[sl]
            pu = plsc.load_gather(p_loc, [u16])
            pv = plsc.load_gather(p_loc, [v16])
            valr = jnp.maximum(pu - pv, 0.0)
            val_buf[sl] = valr
            w = 1.0 + valr
            plsc.addupdate_scatter(dacc, [u16], w)
            plsc.addupdate_scatter(dacc, [v16], w)

    with jax.named_scope("ph_dreduce"):
        for b in range(NTEC):
            pltpu.sync_copy(dacc.at[pl.ds(b * 128, 128)], dstage.at[b, s])
        plsc.subcore_barrier()
        pltpu.sync_copy(dstage.at[s], red_loc)
        for k in range(8):
            acc = red_loc[0, pl.ds(k * 16, 16)]
            for t in range(1, NTEC):
                acc = acc + red_loc[t, pl.ds(k * 16, 16)]
            dred[pl.ds(k * 16, 16)] = _rsqrt16(acc + 1.0)
        pltpu.sync_copy(dred, dinv_sh.at[pl.ds(s * 128, 128)])
        plsc.subcore_barrier()
        pltpu.sync_copy(dinv_sh, dinv_loc)

    for j in range(2):
        chunk = core * 2 + j
        rowbase = chunk * CH

        for t in range(8):
            k16 = iota16 + (t & 1) * 16
            n16 = rowbase + s * DIAG_PER_TEC + k16
            dn = plsc.load_gather(dinv_loc, [n16])
            vv = -dn * dn if t < 2 else zeros16
            sl = pl.ds(2 * ET + t * 16, 16)
            lidx_b[sl] = (s * DIAG_PER_TEC + k16) * N + n16
            val_b[sl] = vv

        gen_scope = jax.named_scope("ph_gen")
        gen_scope.__enter__()

        @pl.loop(0, ET // 16)
        def _(i):
            sl = pl.ds(i * 16, 16)
            u16 = u_buf[sl]
            v16 = v_buf[sl]
            valr = val_buf[sl]
            zero = valr == 0.0
            w = 1.0 + valr
            du = plsc.load_gather(dinv_loc, [u16])
            dv = plsc.load_gather(dinv_loc, [v16])
            duv = du * dv
            for (r16, c16, vv, base) in (
                (u16, jnp.where(zero, u16, v16), -w * jnp.where(zero, du * du, duv), 0),
                (v16, jnp.where(zero, v16, u16), -w * jnp.where(zero, dv * dv, duv), ET),
            ):
                inm = lax.shift_right_logical(r16, 9) == chunk
                e16 = iota16 + i * 16 + base
                lidx = jnp.where(inm, (r16 - rowbase) * N + c16,
                                 SPDUMP + s * 1024 + (e16 & 1023))
                sl2 = pl.ds(base + i * 16, 16)
                lidx_b[sl2] = lidx
                val_b[sl2] = vv

        gen_scope.__exit__(None, None, None)

        rows_off = s * (CH // NTEC) * N
        with jax.named_scope("ph_spzero"):
            pltpu.sync_copy(out_ref.at[pl.ds(2 * NN + rows_off, (CH // NTEC) * N)],
                            spbuf.at[pl.ds(rows_off, (CH // NTEC) * N)])
            plsc.subcore_barrier()
        with jax.named_scope("ph_spadd"):
            pltpu.sync_copy(val_b, spbuf.at[lidx_b], add=True)
            plsc.subcore_barrier()
        with jax.named_scope("ph_out"):
            pltpu.sync_copy(spbuf.at[pl.ds(rows_off, (CH // NTEC) * N)],
                            out_ref.at[pl.ds(NN + rowbase * N + rows_off,
                                             (CH // NTEC) * N)])


def kernel(featuers, Edges, W_e1, b_e1):
    p2 = pl.pallas_call(
        _p_body,
        out_shape=jax.ShapeDtypeStruct((N, 1), jnp.float32),
    )(featuers, W_e1.reshape(1, D), b_e1.reshape(1, 1))
    p = p2.reshape(N)

    init2d = pl.pallas_call(
        _init_body,
        grid=(4 * N // CH - 4,),
        out_specs=pl.BlockSpec((CH, N), lambda i: (jnp.where(i < 4, i, i + 4), 0)),
        out_shape=jax.ShapeDtypeStruct((4 * N, N), jnp.float32),
    )()
    init = init2d.reshape(4 * NN)

    cp = pltpu.CompilerParams()
    if "needs_layout_passes" in pltpu.CompilerParams.__dataclass_fields__:
        cp = dataclasses.replace(cp, needs_layout_passes=False)
    mesh = plsc.VectorSubcoreMesh(core_axis_name="c", subcore_axis_name="s")
    sc_kern = pl.kernel(
        _sc_body,
        out_type=(),
        mesh=mesh,
        compiler_params=cp,
        scratch_types=[
            pltpu.VMEM((ET,), jnp.int32),
            pltpu.VMEM((ET,), jnp.int32),
            pltpu.VMEM((ET,), jnp.float32),
            pltpu.VMEM((N,), jnp.float32),
            pltpu.VMEM((N,), jnp.float32),
            pltpu.VMEM((N,), jnp.float32),
            pltpu.VMEM((NTEC, 128), jnp.float32),
            pltpu.VMEM((128,), jnp.float32),
            pltpu.VMEM((NENT,), jnp.int32),
            pltpu.VMEM((NENT,), jnp.float32),
            pltpu.VMEM_SHARED((NTEC, NTEC, 128), jnp.float32),
            pltpu.VMEM_SHARED((N,), jnp.float32),
            pltpu.VMEM_SHARED((SPDUMP + DUMPW,), jnp.float32),
        ],
    )

    out_ref = jax.new_ref(init)
    sc_kern(Edges.reshape(2 * E), p, out_ref)
    return out_ref[...].reshape(2, 2, N, N)


# speedup vs baseline: 6.2330x; 1.7386x over previous
import dataclasses

import jax
import jax.numpy as jnp
from jax import lax
from jax.experimental import pallas as pl
from jax.experimental.pallas import tpu as pltpu
from jax.experimental.pallas import tpu_sc as plsc

N = 2048
D = 512
E = 32768
NN = N * N
NTEC = 16
ET = E // NTEC
CH = 512
SPDUMP = CH * N
DUMPW = NTEC * 1024
NWIN = 33
NENT = NWIN * 128
DIAG_PER_TEC = CH // NTEC


def _rsqrt16(x):
    i = plsc.bitcast(x, jnp.int32)
    i = 0x5F3759DF - lax.shift_right_logical(i, 1)
    y = plsc.bitcast(i, jnp.float32)
    for _ in range(3):
        y = y * (1.5 - 0.5 * x * y * y)
    return y


def _p_body(x_ref, w_ref, b_ref, o_ref):
    xb = x_ref[...].astype(jnp.bfloat16).astype(jnp.float32)
    wb = w_ref[...].astype(jnp.bfloat16).astype(jnp.float32)
    o_ref[...] = jnp.sum(xb * wb, axis=1, keepdims=True) + b_ref[...]


BR = 128


def _final_body(p1_ref, o_ref):
    i = pl.program_id(0)
    r = lax.broadcasted_iota(jnp.int32, (BR, N), 0) + i * BR
    c = lax.broadcasted_iota(jnp.int32, (BR, N), 1)
    z = jnp.zeros((BR, N), jnp.float32)
    o_ref[0, 0] = jnp.where(r == c, 1.0, 0.0).astype(jnp.float32)
    o_ref[0, 1] = p1_ref[...]
    o_ref[1, 0] = z
    o_ref[1, 1] = z


def _sc_body(edges_ref, p_ref, zeros_ref, out_ref,
             u_buf, v_buf, val_buf, p_loc, dinv_loc, dacc, red_loc, dred,
             lidx_b, val_b,
             dstage, dinv_sh, spbuf):
    core = lax.axis_index("c")
    s = lax.axis_index("s")
    iota16 = lax.iota(jnp.int32, 16)
    zeros16 = jnp.zeros((16,), jnp.float32)

    with jax.named_scope("ph_stage"):
        pltpu.sync_copy(edges_ref.at[pl.ds(s * ET, ET)], u_buf)
        pltpu.sync_copy(edges_ref.at[pl.ds(E + s * ET, ET)], v_buf)
        pltpu.sync_copy(p_ref, p_loc)

        @pl.loop(0, N // 16)
        def _(i):
            dacc[pl.ds(i * 16, 16)] = zeros16

    with jax.named_scope("ph_degree"):
        @pl.loop(0, ET // 16)
        def _(i):
            sl = pl.ds(i * 16, 16)
            u16 = u_buf[sl]
            v16 = v_buf[sl]
            pu = plsc.load_gather(p_loc, [u16])
            pv = plsc.load_gather(p_loc, [v16])
            valr = jnp.maximum(pu - pv, 0.0)
            val_buf[sl] = valr
            w = 1.0 + valr
            plsc.addupdate_scatter(dacc, [u16], w)
            plsc.addupdate_scatter(dacc, [v16], w)

    with jax.named_scope("ph_dreduce"):
        for b in range(NTEC):
            pltpu.sync_copy(dacc.at[pl.ds(b * 128, 128)], dstage.at[b, s])
        plsc.subcore_barrier()
        pltpu.sync_copy(dstage.at[s], red_loc)
        for k in range(8):
            acc = red_loc[0, pl.ds(k * 16, 16)]
            for t in range(1, NTEC):
                acc = acc + red_loc[t, pl.ds(k * 16, 16)]
            dred[pl.ds(k * 16, 16)] = _rsqrt16(acc + 1.0)
        pltpu.sync_copy(dred, dinv_sh.at[pl.ds(s * 128, 128)])
        plsc.subcore_barrier()
        pltpu.sync_copy(dinv_sh, dinv_loc)

    for j in range(2):
        chunk = core * 2 + j
        rowbase = chunk * CH

        for t in range(8):
            k16 = iota16 + (t & 1) * 16
            n16 = rowbase + s * DIAG_PER_TEC + k16
            dn = plsc.load_gather(dinv_loc, [n16])
            vv = -dn * dn if t < 2 else zeros16
            sl = pl.ds(2 * ET + t * 16, 16)
            lidx_b[sl] = (s * DIAG_PER_TEC + k16) * N + n16
            val_b[sl] = vv

        gen_scope = jax.named_scope("ph_gen")
        gen_scope.__enter__()

        @pl.loop(0, ET // 16)
        def _(i):
            sl = pl.ds(i * 16, 16)
            u16 = u_buf[sl]
            v16 = v_buf[sl]
            valr = val_buf[sl]
            zero = valr == 0.0
            w = 1.0 + valr
            du = plsc.load_gather(dinv_loc, [u16])
            dv = plsc.load_gather(dinv_loc, [v16])
            duv = du * dv
            for (r16, c16, vv, base) in (
                (u16, jnp.where(zero, u16, v16), -w * jnp.where(zero, du * du, duv), 0),
                (v16, jnp.where(zero, v16, u16), -w * jnp.where(zero, dv * dv, duv), ET),
            ):
                inm = lax.shift_right_logical(r16, 9) == chunk
                e16 = iota16 + i * 16 + base
                lidx = jnp.where(inm, (r16 - rowbase) * N + c16,
                                 SPDUMP + s * 1024 + (e16 & 1023))
                sl2 = pl.ds(base + i * 16, 16)
                lidx_b[sl2] = lidx
                val_b[sl2] = vv

        gen_scope.__exit__(None, None, None)

        rows_off = s * (CH // NTEC) * N
        with jax.named_scope("ph_spzero"):
            pltpu.sync_copy(zeros_ref.at[pl.ds(rows_off, (CH // NTEC) * N)],
                            spbuf.at[pl.ds(rows_off, (CH // NTEC) * N)])
            plsc.subcore_barrier()
        with jax.named_scope("ph_spadd"):
            pltpu.sync_copy(val_b, spbuf.at[lidx_b], add=True)
            plsc.subcore_barrier()
        with jax.named_scope("ph_out"):
            pltpu.sync_copy(spbuf.at[pl.ds(rows_off, (CH // NTEC) * N)],
                            out_ref.at[pl.ds(rowbase * N + rows_off,
                                             (CH // NTEC) * N)])


def kernel(featuers, Edges, W_e1, b_e1):
    p2 = pl.pallas_call(
        _p_body,
        out_shape=jax.ShapeDtypeStruct((N, 1), jnp.float32),
    )(featuers, W_e1.reshape(1, D), b_e1.reshape(1, 1))
    p = p2.reshape(N)

    cp = pltpu.CompilerParams()
    if "needs_layout_passes" in pltpu.CompilerParams.__dataclass_fields__:
        cp = dataclasses.replace(cp, needs_layout_passes=False)
    mesh = plsc.VectorSubcoreMesh(core_axis_name="c", subcore_axis_name="s")
    sc_kern = pl.kernel(
        _sc_body,
        out_type=jax.ShapeDtypeStruct((NN,), jnp.float32),
        mesh=mesh,
        compiler_params=cp,
        scratch_types=[
            pltpu.VMEM((ET,), jnp.int32),
            pltpu.VMEM((ET,), jnp.int32),
            pltpu.VMEM((ET,), jnp.float32),
            pltpu.VMEM((N,), jnp.float32),
            pltpu.VMEM((N,), jnp.float32),
            pltpu.VMEM((N,), jnp.float32),
            pltpu.VMEM((NTEC, 128), jnp.float32),
            pltpu.VMEM((128,), jnp.float32),
            pltpu.VMEM((NENT,), jnp.int32),
            pltpu.VMEM((NENT,), jnp.float32),
            pltpu.VMEM_SHARED((NTEC, NTEC, 128), jnp.float32),
            pltpu.VMEM_SHARED((N,), jnp.float32),
            pltpu.VMEM_SHARED((SPDUMP + DUMPW,), jnp.float32),
        ],
    )

    zeros_chunk = jnp.zeros((CH * N,), jnp.float32)
    plane1 = sc_kern(Edges.reshape(2 * E), p, zeros_chunk)

    return pl.pallas_call(
        _final_body,
        grid=(N // BR,),
        in_specs=[pl.BlockSpec((BR, N), lambda i: (i, 0))],
        out_specs=pl.BlockSpec((2, 2, BR, N), lambda i: (0, 0, i, 0)),
        out_shape=jax.ShapeDtypeStruct((2, 2, N, N), jnp.float32),
    )(plane1.reshape(N, N))


# speedup vs baseline: 7.4687x; 1.1982x over previous
import dataclasses

import jax
import jax.numpy as jnp
from jax import lax
from jax.experimental import pallas as pl
from jax.experimental.pallas import tpu as pltpu
from jax.experimental.pallas import tpu_sc as plsc

N = 2048
D = 512
E = 32768
NN = N * N
NTEC = 16
ET = E // NTEC
CH = 512
SPDUMP = CH * N
DUMPW = NTEC * 1024
NWIN = 33
NENT = NWIN * 128
DIAG_PER_TEC = CH // NTEC


def _rsqrt16(x):
    i = plsc.bitcast(x, jnp.int32)
    i = 0x5F3759DF - lax.shift_right_logical(i, 1)
    y = plsc.bitcast(i, jnp.float32)
    for _ in range(3):
        y = y * (1.5 - 0.5 * x * y * y)
    return y


def _p_body(x_ref, w_ref, b_ref, o_ref):
    xb = x_ref[...].astype(jnp.bfloat16).astype(jnp.float32)
    wb = w_ref[...].astype(jnp.bfloat16).astype(jnp.float32)
    o_ref[...] = jnp.sum(xb * wb, axis=1, keepdims=True) + b_ref[...]


BR = 128


def _final_body(p1_ref, o_ref):
    i = pl.program_id(0)
    r = lax.broadcasted_iota(jnp.int32, (BR, N), 0) + i * BR
    c = lax.broadcasted_iota(jnp.int32, (BR, N), 1)
    z = jnp.zeros((BR, N), jnp.float32)
    o_ref[0, 0] = jnp.where(r == c, 1.0, 0.0).astype(jnp.float32)
    o_ref[0, 1] = p1_ref[...].reshape(BR, N)
    o_ref[1, 0] = z
    o_ref[1, 1] = z


def _sc_body(edges_ref, p_ref, zeros_ref, out_ref,
             u_buf, v_buf, val_buf, p_loc, dinv_loc, dacc, red_loc, dred,
             lidx_b, val_b,
             dstage, dinv_sh, spbuf):
    core = lax.axis_index("c")
    s = lax.axis_index("s")
    iota16 = lax.iota(jnp.int32, 16)
    zeros16 = jnp.zeros((16,), jnp.float32)

    with jax.named_scope("ph_stage"):
        pltpu.sync_copy(edges_ref.at[pl.ds(s * ET, ET)], u_buf)
        pltpu.sync_copy(edges_ref.at[pl.ds(E + s * ET, ET)], v_buf)
        pltpu.sync_copy(p_ref, p_loc)

        @pl.loop(0, N // 16)
        def _(i):
            dacc[pl.ds(i * 16, 16)] = zeros16

    with jax.named_scope("ph_degree"):
        @pl.loop(0, ET // 16)
        def _(i):
            sl = pl.ds(i * 16, 16)
            u16 = u_buf[sl]
            v16 = v_buf[sl]
            pu = plsc.load_gather(p_loc, [u16])
            pv = plsc.load_gather(p_loc, [v16])
            valr = jnp.maximum(pu - pv, 0.0)
            val_buf[sl] = valr
            w = 1.0 + valr
            plsc.addupdate_scatter(dacc, [u16], w)
            plsc.addupdate_scatter(dacc, [v16], w)

    with jax.named_scope("ph_dreduce"):
        for b in range(NTEC):
            pltpu.sync_copy(dacc.at[pl.ds(b * 128, 128)], dstage.at[b, s])
        plsc.subcore_barrier()
        pltpu.sync_copy(dstage.at[s], red_loc)
        for k in range(8):
            acc = red_loc[0, pl.ds(k * 16, 16)]
            for t in range(1, NTEC):
                acc = acc + red_loc[t, pl.ds(k * 16, 16)]
            dred[pl.ds(k * 16, 16)] = _rsqrt16(acc + 1.0)
        pltpu.sync_copy(dred, dinv_sh.at[pl.ds(s * 128, 128)])
        plsc.subcore_barrier()
        pltpu.sync_copy(dinv_sh, dinv_loc)

    for j in range(2):
        chunk = core * 2 + j
        rowbase = chunk * CH

        for t in range(8):
            k16 = iota16 + (t & 1) * 16
            n16 = rowbase + s * DIAG_PER_TEC + k16
            dn = plsc.load_gather(dinv_loc, [n16])
            vv = -dn * dn if t < 2 else zeros16
            sl = pl.ds(2 * ET + t * 16, 16)
            lidx_b[sl] = (s * DIAG_PER_TEC + k16) * N + n16
            val_b[sl] = vv

        gen_scope = jax.named_scope("ph_gen")
        gen_scope.__enter__()

        @pl.loop(0, ET // 16)
        def _(i):
            sl = pl.ds(i * 16, 16)
            u16 = u_buf[sl]
            v16 = v_buf[sl]
            valr = val_buf[sl]
            zero = valr == 0.0
            w = 1.0 + valr
            du = plsc.load_gather(dinv_loc, [u16])
            dv = plsc.load_gather(dinv_loc, [v16])
            duv = du * dv
            for (r16, c16, vv, base) in (
                (u16, jnp.where(zero, u16, v16), -w * jnp.where(zero, du * du, duv), 0),
                (v16, jnp.where(zero, v16, u16), -w * jnp.where(zero, dv * dv, duv), ET),
            ):
                inm = lax.shift_right_logical(r16, 9) == chunk
                e16 = iota16 + i * 16 + base
                lidx = jnp.where(inm, (r16 - rowbase) * N + c16,
                                 SPDUMP + s * 1024 + (e16 & 1023))
                sl2 = pl.ds(base + i * 16, 16)
                lidx_b[sl2] = lidx
                val_b[sl2] = vv

        gen_scope.__exit__(None, None, None)

        rows_off = s * (CH // NTEC) * N
        with jax.named_scope("ph_spzero"):
            pltpu.sync_copy(zeros_ref.at[pl.ds(rows_off, (CH // NTEC) * N)],
                            spbuf.at[pl.ds(rows_off, (CH // NTEC) * N)])
            plsc.subcore_barrier()
        with jax.named_scope("ph_spadd"):
            pltpu.sync_copy(val_b, spbuf.at[lidx_b], add=True)
            plsc.subcore_barrier()
        with jax.named_scope("ph_out"):
            pltpu.sync_copy(spbuf.at[pl.ds(rows_off, (CH // NTEC) * N)],
                            out_ref.at[pl.ds(rowbase * N + rows_off,
                                             (CH // NTEC) * N)])


def kernel(featuers, Edges, W_e1, b_e1):
    p2 = pl.pallas_call(
        _p_body,
        out_shape=jax.ShapeDtypeStruct((N, 1), jnp.float32),
    )(featuers, W_e1.reshape(1, D), b_e1.reshape(1, 1))
    p = p2.reshape(N)

    cp = pltpu.CompilerParams()
    if "needs_layout_passes" in pltpu.CompilerParams.__dataclass_fields__:
        cp = dataclasses.replace(cp, needs_layout_passes=False)
    mesh = plsc.VectorSubcoreMesh(core_axis_name="c", subcore_axis_name="s")
    sc_kern = pl.kernel(
        _sc_body,
        out_type=jax.ShapeDtypeStruct((NN,), jnp.float32),
        mesh=mesh,
        compiler_params=cp,
        scratch_types=[
            pltpu.VMEM((ET,), jnp.int32),
            pltpu.VMEM((ET,), jnp.int32),
            pltpu.VMEM((ET,), jnp.float32),
            pltpu.VMEM((N,), jnp.float32),
            pltpu.VMEM((N,), jnp.float32),
            pltpu.VMEM((N,), jnp.float32),
            pltpu.VMEM((NTEC, 128), jnp.float32),
            pltpu.VMEM((128,), jnp.float32),
            pltpu.VMEM((NENT,), jnp.int32),
            pltpu.VMEM((NENT,), jnp.float32),
            pltpu.VMEM_SHARED((NTEC, NTEC, 128), jnp.float32),
            pltpu.VMEM_SHARED((N,), jnp.float32),
            pltpu.VMEM_SHARED((SPDUMP + DUMPW,), jnp.float32),
        ],
    )

    zeros_chunk = jnp.zeros((CH * N,), jnp.float32)
    plane1 = sc_kern(Edges.reshape(2 * E), p, zeros_chunk)

    return pl.pallas_call(
        _final_body,
        grid=(N // BR,),
        in_specs=[pl.BlockSpec((BR * N,), lambda i: (i,))],
        out_specs=pl.BlockSpec((2, 2, BR, N), lambda i: (0, 0, i, 0)),
        out_shape=jax.ShapeDtypeStruct((2, 2, N, N), jnp.float32),
    )(plane1)


# speedup vs baseline: 7.9557x; 1.0652x over previous
import dataclasses

import jax
import jax.numpy as jnp
from jax import lax
from jax.experimental import pallas as pl
from jax.experimental.pallas import tpu as pltpu
from jax.experimental.pallas import tpu_sc as plsc

N = 2048
D = 512
E = 32768
NN = N * N
NTEC = 16
ET = E // NTEC
CH = 512
SPDUMP = CH * N
DUMPW = NTEC * 1024
NWIN = 33
NENT = NWIN * 128
DIAG_PER_TEC = CH // NTEC


def _rsqrt16(x):
    i = plsc.bitcast(x, jnp.int32)
    i = 0x5F3759DF - lax.shift_right_logical(i, 1)
    y = plsc.bitcast(i, jnp.float32)
    for _ in range(3):
        y = y * (1.5 - 0.5 * x * y * y)
    return y


def _p_body(x_ref, w_ref, b_ref, o_ref):
    xb = x_ref[...].astype(jnp.bfloat16).astype(jnp.float32)
    wb = w_ref[...].astype(jnp.bfloat16).astype(jnp.float32)
    o_ref[...] = jnp.sum(xb * wb, axis=1, keepdims=True) + b_ref[...]


BR = 256


def _final_body(p1_ref, o_ref):
    i = pl.program_id(0)
    r = lax.broadcasted_iota(jnp.int32, (BR, N), 0) + i * BR
    c = lax.broadcasted_iota(jnp.int32, (BR, N), 1)
    z = jnp.zeros((BR, N), jnp.float32)
    o_ref[0, 0] = jnp.where(r == c, 1.0, 0.0).astype(jnp.float32)
    o_ref[0, 1] = p1_ref[...].reshape(BR, N)
    o_ref[1, 0] = z
    o_ref[1, 1] = z


def _sc_body(edges_ref, p_ref, zeros_ref, out_ref,
             u_buf, v_buf, val_buf, p_loc, dinv_loc, dacc, red_loc, dred,
             lidx_b0, val_b0, lidx_b1, val_b1, z_b,
             dstage, dinv_sh, spbuf, sem_z):
    core = lax.axis_index("c")
    s = lax.axis_index("s")
    iota16 = lax.iota(jnp.int32, 16)
    zeros16 = jnp.zeros((16,), jnp.float32)
    rows_off = s * (CH // NTEC) * N
    rows_w = (CH // NTEC) * N

    zin = pltpu.async_copy(zeros_ref.at[pl.ds(rows_off, rows_w)],
                           spbuf.at[pl.ds(rows_off, rows_w)], sem_z)

    with jax.named_scope("ph_stage"):
        pltpu.sync_copy(edges_ref.at[pl.ds(s * ET, ET)], u_buf)
        pltpu.sync_copy(edges_ref.at[pl.ds(E + s * ET, ET)], v_buf)
        pltpu.sync_copy(p_ref, p_loc)

        @pl.loop(0, N // 16)
        def _(i):
            dacc[pl.ds(i * 16, 16)] = zeros16

        @pl.loop(0, NENT // 16)
        def _(i):
            z_b[pl.ds(i * 16, 16)] = zeros16

    with jax.named_scope("ph_degree"):
        @pl.loop(0, ET // 16)
        def _(i):
            sl = pl.ds(i * 16, 16)
            u16 = u_buf[sl]
            v16 = v_buf[sl]
            pu = plsc.load_gather(p_loc, [u16])
            pv = plsc.load_gather(p_loc, [v16])
            valr = jnp.maximum(pu - pv, 0.0)
            val_buf[sl] = valr
            w = 1.0 + valr
            plsc.addupdate_scatter(dacc, [u16], w)
            plsc.addupdate_scatter(dacc, [v16], w)

    with jax.named_scope("ph_dreduce"):
        for b in range(NTEC):
            pltpu.sync_copy(dacc.at[pl.ds(b * 128, 128)], dstage.at[b, s])
        plsc.subcore_barrier()
        pltpu.sync_copy(dstage.at[s], red_loc)
        for k in range(8):
            acc = red_loc[0, pl.ds(k * 16, 16)]
            for t in range(1, NTEC):
                acc = acc + red_loc[t, pl.ds(k * 16, 16)]
            dred[pl.ds(k * 16, 16)] = _rsqrt16(acc + 1.0)
        pltpu.sync_copy(dred, dinv_sh.at[pl.ds(s * 128, 128)])
        plsc.subcore_barrier()
        pltpu.sync_copy(dinv_sh, dinv_loc)

    for j, (lidx_b, val_b) in enumerate(((lidx_b0, val_b0), (lidx_b1, val_b1))):
        chunk = core * 2 + j
        rowbase = chunk * CH

        for t in range(8):
            k16 = iota16 + (t & 1) * 16
            n16 = rowbase + s * DIAG_PER_TEC + k16
            dn = plsc.load_gather(dinv_loc, [n16])
            vv = -dn * dn if t < 2 else zeros16
            sl = pl.ds(2 * ET + t * 16, 16)
            lidx_b[sl] = (s * DIAG_PER_TEC + k16) * N + n16
            val_b[sl] = vv

        gen_scope = jax.named_scope("ph_gen")
        gen_scope.__enter__()

        @pl.loop(0, ET // 16)
        def _(i):
            sl = pl.ds(i * 16, 16)
            u16 = u_buf[sl]
            v16 = v_buf[sl]
            valr = val_buf[sl]
            zero = valr == 0.0
            w = 1.0 + valr
            du = plsc.load_gather(dinv_loc, [u16])
            dv = plsc.load_gather(dinv_loc, [v16])
            duv = du * dv
            for (r16, c16, vv, base) in (
                (u16, jnp.where(zero, u16, v16), -w * jnp.where(zero, du * du, duv), 0),
                (v16, jnp.where(zero, v16, u16), -w * jnp.where(zero, dv * dv, duv), ET),
            ):
                inm = lax.shift_right_logical(r16, 9) == chunk
                e16 = iota16 + i * 16 + base
                lidx = jnp.where(inm, (r16 - rowbase) * N + c16,
                                 SPDUMP + s * 1024 + (e16 & 1023))
                sl2 = pl.ds(base + i * 16, 16)
                lidx_b[sl2] = lidx
                val_b[sl2] = vv

        gen_scope.__exit__(None, None, None)

    with jax.named_scope("ph_spzero"):
        zin.wait()
        plsc.subcore_barrier()
    with jax.named_scope("ph_spadd"):
        pltpu.sync_copy(val_b0, spbuf.at[lidx_b0], add=True)
        plsc.subcore_barrier()
    with jax.named_scope("ph_out"):
        pltpu.sync_copy(spbuf.at[pl.ds(rows_off, rows_w)],
                        out_ref.at[pl.ds(core * 2 * CH * N + rows_off, rows_w)])
        plsc.subcore_barrier()

    with jax.named_scope("ph_spzero"):
        pltpu.sync_copy(z_b, spbuf.at[lidx_b0])
        plsc.subcore_barrier()
    with jax.named_scope("ph_spadd"):
        pltpu.sync_copy(val_b1, spbuf.at[lidx_b1], add=True)
        plsc.subcore_barrier()
    with jax.named_scope("ph_out"):
        pltpu.sync_copy(spbuf.at[pl.ds(rows_off, rows_w)],
                        out_ref.at[pl.ds((core * 2 + 1) * CH * N + rows_off, rows_w)])


def kernel(featuers, Edges, W_e1, b_e1):
    p2 = pl.pallas_call(
        _p_body,
        out_shape=jax.ShapeDtypeStruct((N, 1), jnp.float32),
    )(featuers, W_e1.reshape(1, D), b_e1.reshape(1, 1))
    p = p2.reshape(N)

    cp = pltpu.CompilerParams()
    if "needs_layout_passes" in pltpu.CompilerParams.__dataclass_fields__:
        cp = dataclasses.replace(cp, needs_layout_passes=False)
    mesh = plsc.VectorSubcoreMesh(core_axis_name="c", subcore_axis_name="s")
    sc_kern = pl.kernel(
        _sc_body,
        out_type=jax.ShapeDtypeStruct((NN,), jnp.float32),
        mesh=mesh,
        compiler_params=cp,
        scratch_types=[
            pltpu.VMEM((ET,), jnp.int32),
            pltpu.VMEM((ET,), jnp.int32),
            pltpu.VMEM((ET,), jnp.float32),
            pltpu.VMEM((N,), jnp.float32),
            pltpu.VMEM((N,), jnp.float32),
            pltpu.VMEM((N,), jnp.float32),
            pltpu.VMEM((NTEC, 128), jnp.float32),
            pltpu.VMEM((128,), jnp.float32),
            pltpu.VMEM((NENT,), jnp.int32),
            pltpu.VMEM((NENT,), jnp.float32),
            pltpu.VMEM((NENT,), jnp.int32),
            pltpu.VMEM((NENT,), jnp.float32),
            pltpu.VMEM((NENT,), jnp.float32),
            pltpu.VMEM_SHARED((NTEC, NTEC, 128), jnp.float32),
            pltpu.VMEM_SHARED((N,), jnp.float32),
            pltpu.VMEM_SHARED((SPDUMP + DUMPW,), jnp.float32),
            pltpu.SemaphoreType.DMA,
        ],
    )

    zeros_chunk = jnp.zeros((CH * N,), jnp.float32)
    plane1 = sc_kern(Edges.reshape(2 * E), p, zeros_chunk)

    return pl.pallas_call(
        _final_body,
        grid=(N // BR,),
        in_specs=[pl.BlockSpec((BR * N,), lambda i: (i,))],
        out_specs=pl.BlockSpec((2, 2, BR, N), lambda i: (0, 0, i, 0)),
        out_shape=jax.ShapeDtypeStruct((2, 2, N, N), jnp.float32),
    )(plane1)


# speedup vs baseline: 8.0639x; 1.0136x over previous
import dataclasses

import jax
import jax.numpy as jnp
from jax import lax
from jax.experimental import pallas as pl
from jax.experimental.pallas import tpu as pltpu
from jax.experimental.pallas import tpu_sc as plsc

N = 2048
D = 512
E = 32768
NN = N * N
NTEC = 16
ET = E // NTEC
CH = 512
SPDUMP = CH * N
DUMPW = NTEC * 1024
NWIN = 33
NENT = NWIN * 128
DIAG_PER_TEC = CH // NTEC


def _rsqrt16(x):
    i = plsc.bitcast(x, jnp.int32)
    i = 0x5F3759DF - lax.shift_right_logical(i, 1)
    y = plsc.bitcast(i, jnp.float32)
    for _ in range(3):
        y = y * (1.5 - 0.5 * x * y * y)
    return y


def _p_body(x_ref, w_ref, b_ref, o_ref):
    xb = x_ref[...].astype(jnp.bfloat16).astype(jnp.float32)
    wb = w_ref[...].astype(jnp.bfloat16).astype(jnp.float32)
    o_ref[...] = jnp.sum(xb * wb, axis=1, keepdims=True) + b_ref[...]


BR = 512


def _final_body(p1_ref, o_ref):
    i = pl.program_id(0)
    r = lax.broadcasted_iota(jnp.int32, (BR, N), 0) + i * BR
    c = lax.broadcasted_iota(jnp.int32, (BR, N), 1)
    z = jnp.zeros((BR, N), jnp.float32)
    o_ref[0, 0] = jnp.where(r == c, 1.0, 0.0).astype(jnp.float32)
    o_ref[0, 1] = p1_ref[...].reshape(BR, N)
    o_ref[1, 0] = z
    o_ref[1, 1] = z


def _sc_body(u_ref, v_ref, p_ref, zeros_ref, out_ref,
             u_buf, v_buf, val_buf, p_loc, dinv_loc, dacc, red_loc, dred,
             lidx_b0, val_b0, lidx_b1, val_b1, z_b,
             dstage, dinv_sh, spbuf, sem_z):
    core = lax.axis_index("c")
    s = lax.axis_index("s")
    iota16 = lax.iota(jnp.int32, 16)
    zeros16 = jnp.zeros((16,), jnp.float32)
    rows_off = s * (CH // NTEC) * N
    rows_w = (CH // NTEC) * N

    zin = pltpu.async_copy(zeros_ref,
                           spbuf.at[pl.ds(rows_off, rows_w)], sem_z)

    with jax.named_scope("ph_stage"):
        pltpu.sync_copy(u_ref.at[pl.ds(s * ET, ET)], u_buf)
        pltpu.sync_copy(v_ref.at[pl.ds(s * ET, ET)], v_buf)
        pltpu.sync_copy(p_ref, p_loc)

        @pl.loop(0, N // 16)
        def _(i):
            dacc[pl.ds(i * 16, 16)] = zeros16

        @pl.loop(0, NENT // 16)
        def _(i):
            z_b[pl.ds(i * 16, 16)] = zeros16

    with jax.named_scope("ph_degree"):
        @pl.loop(0, ET // 16)
        def _(i):
            sl = pl.ds(i * 16, 16)
            u16 = u_buf[sl]
            v16 = v_buf[sl]
            pu = plsc.load_gather(p_loc, [u16])
            pv = plsc.load_gather(p_loc, [v16])
            valr = jnp.maximum(pu - pv, 0.0)
            val_buf[sl] = valr
            w = 1.0 + valr
            plsc.addupdate_scatter(dacc, [u16], w)
            plsc.addupdate_scatter(dacc, [v16], w)

    with jax.named_scope("ph_dreduce"):
        for b in range(NTEC):
            pltpu.sync_copy(dacc.at[pl.ds(b * 128, 128)], dstage.at[b, s])
        plsc.subcore_barrier()
        pltpu.sync_copy(dstage.at[s], red_loc)
        for k in range(8):
            acc = red_loc[0, pl.ds(k * 16, 16)]
            for t in range(1, NTEC):
                acc = acc + red_loc[t, pl.ds(k * 16, 16)]
            dred[pl.ds(k * 16, 16)] = _rsqrt16(acc + 1.0)
        pltpu.sync_copy(dred, dinv_sh.at[pl.ds(s * 128, 128)])
        plsc.subcore_barrier()
        pltpu.sync_copy(dinv_sh, dinv_loc)

    for j, (lidx_b, val_b) in enumerate(((lidx_b0, val_b0), (lidx_b1, val_b1))):
        chunk = core * 2 + j
        rowbase = chunk * CH

        for t in range(8):
            k16 = iota16 + (t & 1) * 16
            n16 = rowbase + s * DIAG_PER_TEC + k16
            dn = plsc.load_gather(dinv_loc, [n16])
            vv = -dn * dn if t < 2 else zeros16
            sl = pl.ds(2 * ET + t * 16, 16)
            lidx_b[sl] = (s * DIAG_PER_TEC + k16) * N + n16
            val_b[sl] = vv

        gen_scope = jax.named_scope("ph_gen")
        gen_scope.__enter__()

        @pl.loop(0, ET // 16)
        def _(i):
            sl = pl.ds(i * 16, 16)
            u16 = u_buf[sl]
            v16 = v_buf[sl]
            valr = val_buf[sl]
            zero = valr == 0.0
            w = 1.0 + valr
            du = plsc.load_gather(dinv_loc, [u16])
            dv = plsc.load_gather(dinv_loc, [v16])
            duv = du * dv
            for (r16, c16, vv, base) in (
                (u16, jnp.where(zero, u16, v16), -w * jnp.where(zero, du * du, duv), 0),
                (v16, jnp.where(zero, v16, u16), -w * jnp.where(zero, dv * dv, duv), ET),
            ):
                inm = lax.shift_right_logical(r16, 9) == chunk
                e16 = iota16 + i * 16 + base
                lidx = jnp.where(inm, (r16 - rowbase) * N + c16,
                                 SPDUMP + s * 1024 + (e16 & 1023))
                sl2 = pl.ds(base + i * 16, 16)
                lidx_b[sl2] = lidx
                val_b[sl2] = vv

        gen_scope.__exit__(None, None, None)

    with jax.named_scope("ph_spzero"):
        zin.wait()
        plsc.subcore_barrier()
    with jax.named_scope("ph_spadd"):
        pltpu.sync_copy(val_b0, spbuf.at[lidx_b0], add=True)
        plsc.subcore_barrier()
    with jax.named_scope("ph_out"):
        pltpu.sync_copy(spbuf.at[pl.ds(rows_off, rows_w)],
                        out_ref.at[pl.ds(core * 2 * CH * N + rows_off, rows_w)])
        plsc.subcore_barrier()

    with jax.named_scope("ph_spzero"):
        pltpu.sync_copy(z_b, spbuf.at[lidx_b0])
        plsc.subcore_barrier()
    with jax.named_scope("ph_spadd"):
        pltpu.sync_copy(val_b1, spbuf.at[lidx_b1], add=True)
        plsc.subcore_barrier()
    with jax.named_scope("ph_out"):
        pltpu.sync_copy(spbuf.at[pl.ds(rows_off, rows_w)],
                        out_ref.at[pl.ds((core * 2 + 1) * CH * N + rows_off, rows_w)])


def kernel(featuers, Edges, W_e1, b_e1):
    p2 = pl.pallas_call(
        _p_body,
        out_shape=jax.ShapeDtypeStruct((N, 1), jnp.float32),
    )(featuers, W_e1.reshape(1, D), b_e1.reshape(1, 1))

    cp = pltpu.CompilerParams()
    if "needs_layout_passes" in pltpu.CompilerParams.__dataclass_fields__:
        cp = dataclasses.replace(cp, needs_layout_passes=False)
    mesh = plsc.VectorSubcoreMesh(core_axis_name="c", subcore_axis_name="s")
    sc_kern = pl.kernel(
        _sc_body,
        out_type=jax.ShapeDtypeStruct((NN,), jnp.float32),
        mesh=mesh,
        compiler_params=cp,
        scratch_types=[
            pltpu.VMEM((ET,), jnp.int32),
            pltpu.VMEM((ET,), jnp.int32),
            pltpu.VMEM((ET,), jnp.float32),
            pltpu.VMEM((N,), jnp.float32),
            pltpu.VMEM((N,), jnp.float32),
            pltpu.VMEM((N,), jnp.float32),
            pltpu.VMEM((NTEC, 128), jnp.float32),
            pltpu.VMEM((128,), jnp.float32),
            pltpu.VMEM((NENT,), jnp.int32),
            pltpu.VMEM((NENT,), jnp.float32),
            pltpu.VMEM((NENT,), jnp.int32),
            pltpu.VMEM((NENT,), jnp.float32),
            pltpu.VMEM((NENT,), jnp.float32),
            pltpu.VMEM_SHARED((NTEC, NTEC, 128), jnp.float32),
            pltpu.VMEM_SHARED((N,), jnp.float32),
            pltpu.VMEM_SHARED((SPDUMP + DUMPW,), jnp.float32),
            pltpu.SemaphoreType.DMA,
        ],
    )

    zeros_chunk = jnp.zeros(((CH // NTEC) * N,), jnp.float32)
    plane1 = sc_kern(Edges[0], Edges[1], p2.reshape(N), zeros_chunk)

    return pl.pallas_call(
        _final_body,
        grid=(N // BR,),
        in_specs=[pl.BlockSpec((BR * N,), lambda i: (i,))],
        out_specs=pl.BlockSpec((2, 2, BR, N), lambda i: (0, 0, i, 0)),
        out_shape=jax.ShapeDtypeStruct((2, 2, N, N), jnp.float32),
    )(plane1)


# speedup vs baseline: 8.3256x; 1.0325x over previous
import dataclasses

import jax
import jax.numpy as jnp
from jax import lax
from jax.experimental import pallas as pl
from jax.experimental.pallas import tpu as pltpu
from jax.experimental.pallas import tpu_sc as plsc

N = 2048
D = 512
E = 32768
NN = N * N
NTEC = 16
ET = E // NTEC
CH = 512
SPDUMP = CH * N
DUMPW = NTEC * 1024
NWIN = 33
NENT = NWIN * 128
DIAG_PER_TEC = CH // NTEC


def _rsqrt16(x):
    i = plsc.bitcast(x, jnp.int32)
    i = 0x5F3759DF - lax.shift_right_logical(i, 1)
    y = plsc.bitcast(i, jnp.float32)
    for _ in range(3):
        y = y * (1.5 - 0.5 * x * y * y)
    return y


def _p_body(x_ref, w_ref, b_ref, o_ref):
    xb = x_ref[...].astype(jnp.bfloat16).astype(jnp.float32)
    wb = w_ref[...].astype(jnp.bfloat16).astype(jnp.float32)
    o_ref[...] = jnp.sum(xb * wb, axis=1) + b_ref[0, 0]


BR = 512


def _final_body(p1_ref, o_ref):
    i = pl.program_id(0)
    r = lax.broadcasted_iota(jnp.int32, (BR, N), 0) + i * BR
    c = lax.broadcasted_iota(jnp.int32, (BR, N), 1)
    z = jnp.zeros((BR, N), jnp.float32)
    o_ref[0, 0] = jnp.where(r == c, 1.0, 0.0).astype(jnp.float32)
    o_ref[0, 1] = p1_ref[...].reshape(BR, N)
    o_ref[1, 0] = z
    o_ref[1, 1] = z


def _sc_body(u_ref, v_ref, p_ref, zeros_ref, out_ref,
             u_buf, v_buf, val_buf, p_loc, dinv_loc, dacc, red_loc, dred,
             lidx_b0, val_b0, lidx_b1, val_b1, z_b,
             dstage, dinv_sh, spbuf, sem_z):
    core = lax.axis_index("c")
    s = lax.axis_index("s")
    iota16 = lax.iota(jnp.int32, 16)
    zeros16 = jnp.zeros((16,), jnp.float32)
    rows_off = s * (CH // NTEC) * N
    rows_w = (CH // NTEC) * N

    with jax.named_scope("ph_stage"):
        pltpu.sync_copy(u_ref.at[pl.ds(s * ET, ET)], u_buf)
        pltpu.sync_copy(v_ref.at[pl.ds(s * ET, ET)], v_buf)
        pltpu.sync_copy(p_ref, p_loc)
        zin = pltpu.async_copy(zeros_ref,
                               spbuf.at[pl.ds(rows_off, rows_w)], sem_z)

        @pl.loop(0, N // 16)
        def _(i):
            dacc[pl.ds(i * 16, 16)] = zeros16

        @pl.loop(0, NENT // 16)
        def _(i):
            z_b[pl.ds(i * 16, 16)] = zeros16

    with jax.named_scope("ph_degree"):
        @pl.loop(0, ET // 16)
        def _(i):
            sl = pl.ds(i * 16, 16)
            u16 = u_buf[sl]
            v16 = v_buf[sl]
            pu = plsc.load_gather(p_loc, [u16])
            pv = plsc.load_gather(p_loc, [v16])
            valr = jnp.maximum(pu - pv, 0.0)
            val_buf[sl] = valr
            w = 1.0 + valr
            plsc.addupdate_scatter(dacc, [u16], w)
            plsc.addupdate_scatter(dacc, [v16], w)

    with jax.named_scope("ph_dreduce"):
        for b in range(NTEC):
            pltpu.sync_copy(dacc.at[pl.ds(b * 128, 128)], dstage.at[b, s])
        plsc.subcore_barrier()
        pltpu.sync_copy(dstage.at[s], red_loc)
        for k in range(8):
            acc = red_loc[0, pl.ds(k * 16, 16)]
            for t in range(1, NTEC):
                acc = acc + red_loc[t, pl.ds(k * 16, 16)]
            dred[pl.ds(k * 16, 16)] = _rsqrt16(acc + 1.0)
        pltpu.sync_copy(dred, dinv_sh.at[pl.ds(s * 128, 128)])
        plsc.subcore_barrier()
        pltpu.sync_copy(dinv_sh, dinv_loc)

    for j, (lidx_b, val_b) in enumerate(((lidx_b0, val_b0), (lidx_b1, val_b1))):
        chunk = core * 2 + j
        rowbase = chunk * CH

        for t in range(8):
            k16 = iota16 + (t & 1) * 16
            n16 = rowbase + s * DIAG_PER_TEC + k16
            dn = plsc.load_gather(dinv_loc, [n16])
            vv = -dn * dn if t < 2 else zeros16
            sl = pl.ds(2 * ET + t * 16, 16)
            lidx_b[sl] = (s * DIAG_PER_TEC + k16) * N + n16
            val_b[sl] = vv

        gen_scope = jax.named_scope("ph_gen")
        gen_scope.__enter__()

        @pl.loop(0, ET // 16)
        def _(i):
            sl = pl.ds(i * 16, 16)
            u16 = u_buf[sl]
            v16 = v_buf[sl]
            valr = val_buf[sl]
            zero = valr == 0.0
            w = 1.0 + valr
            du = plsc.load_gather(dinv_loc, [u16])
            dv = plsc.load_gather(dinv_loc, [v16])
            duv = du * dv
            for (r16, c16, vv, base) in (
                (u16, jnp.where(zero, u16, v16), -w * jnp.where(zero, du * du, duv), 0),
                (v16, jnp.where(zero, v16, u16), -w * jnp.where(zero, dv * dv, duv), ET),
            ):
                inm = lax.shift_right_logical(r16, 9) == chunk
                e16 = iota16 + i * 16 + base
                lidx = jnp.where(inm, (r16 - rowbase) * N + c16,
                                 SPDUMP + s * 1024 + (e16 & 1023))
                sl2 = pl.ds(base + i * 16, 16)
                lidx_b[sl2] = lidx
                val_b[sl2] = vv

        gen_scope.__exit__(None, None, None)

    with jax.named_scope("ph_spzero"):
        zin.wait()
        plsc.subcore_barrier()
    with jax.named_scope("ph_spadd"):
        pltpu.sync_copy(val_b0, spbuf.at[lidx_b0], add=True)
        plsc.subcore_barrier()
    with jax.named_scope("ph_out"):
        pltpu.sync_copy(spbuf.at[pl.ds(rows_off, rows_w)],
                        out_ref.at[pl.ds(core * 2 * CH * N + rows_off, rows_w)])
        plsc.subcore_barrier()

    with jax.named_scope("ph_spzero"):
        pltpu.sync_copy(z_b, spbuf.at[lidx_b0])
        plsc.subcore_barrier()
    with jax.named_scope("ph_spadd"):
        pltpu.sync_copy(val_b1, spbuf.at[lidx_b1], add=True)
        plsc.subcore_barrier()
    with jax.named_scope("ph_out"):
        pltpu.sync_copy(spbuf.at[pl.ds(rows_off, rows_w)],
                        out_ref.at[pl.ds((core * 2 + 1) * CH * N + rows_off, rows_w)])


def kernel(featuers, Edges, W_e1, b_e1):
    p2 = pl.pallas_call(
        _p_body,
        out_shape=jax.ShapeDtypeStruct((N,), jnp.float32),
    )(featuers, W_e1.reshape(1, D), b_e1.reshape(1, 1))

    cp = pltpu.CompilerParams()
    if "needs_layout_passes" in pltpu.CompilerParams.__dataclass_fields__:
        cp = dataclasses.replace(cp, needs_layout_passes=False)
    mesh = plsc.VectorSubcoreMesh(core_axis_name="c", subcore_axis_name="s")
    sc_kern = pl.kernel(
        _sc_body,
        out_type=jax.ShapeDtypeStruct((NN,), jnp.float32),
        mesh=mesh,
        compiler_params=cp,
        scratch_types=[
            pltpu.VMEM((ET,), jnp.int32),
            pltpu.VMEM((ET,), jnp.int32),
            pltpu.VMEM((ET,), jnp.float32),
            pltpu.VMEM((N,), jnp.float32),
            pltpu.VMEM((N,), jnp.float32),
            pltpu.VMEM((N,), jnp.float32),
            pltpu.VMEM((NTEC, 128), jnp.float32),
            pltpu.VMEM((128,), jnp.float32),
            pltpu.VMEM((NENT,), jnp.int32),
            pltpu.VMEM((NENT,), jnp.float32),
            pltpu.VMEM((NENT,), jnp.int32),
            pltpu.VMEM((NENT,), jnp.float32),
            pltpu.VMEM((NENT,), jnp.float32),
            pltpu.VMEM_SHARED((NTEC, NTEC, 128), jnp.float32),
            pltpu.VMEM_SHARED((N,), jnp.float32),
            pltpu.VMEM_SHARED((SPDUMP + DUMPW,), jnp.float32),
            pltpu.SemaphoreType.DMA,
        ],
    )

    zeros_chunk = jnp.zeros(((CH // NTEC) * N,), jnp.float32)
    plane1 = sc_kern(Edges[0], Edges[1], p2, zeros_chunk)

    return pl.pallas_call(
        _final_body,
        grid=(N // BR,),
        in_specs=[pl.BlockSpec((BR * N,), lambda i: (i,))],
        out_specs=pl.BlockSpec((2, 2, BR, N), lambda i: (0, 0, i, 0)),
        out_shape=jax.ShapeDtypeStruct((2, 2, N, N), jnp.float32),
    )(plane1)
